# baseline (speedup 1.0000x reference)
import functools

import jax
import jax.numpy as jnp
from jax import lax
from jax.experimental import pallas as pl
from jax.experimental.pallas import tpu as pltpu
from jax.experimental.pallas import tpu_sc as plsc

N = 10000
D = 128
E = 320000
NC, NS, L = 2, 16, 16
NW = NC * NS
NPAD = 10240
CH = 128
TOT_CH = 2560
SUP = 40
NBUF = 2
EPAD = TOT_CH * CH
CPT0 = 80
CPT1 = 80
RPT = NPAD // NS
RB = 1024


def _make_sc_agg(compute_deg: bool):
    mesh = plsc.VectorSubcoreMesh(core_axis_name="c", subcore_axis_name="s")
    part = jax.ShapeDtypeStruct((NC, NPAD, D), jnp.float32)
    out_type = [part]
    scratch = (
        [pltpu.VMEM((SUP, CH), jnp.int32)] * 2
        + [pltpu.VMEM((CH, D), jnp.float32)] * NBUF
        + [pltpu.VMEM_SHARED((NPAD, D), jnp.float32)]
        + [pltpu.SemaphoreType.DMA] * NBUF
    )
    if compute_deg:
        out_type.append(jax.ShapeDtypeStruct((NC, NPAD), jnp.float32))
        scratch.append(pltpu.VMEM((CH,), jnp.float32))
        scratch.append(pltpu.VMEM_SHARED((NPAD,), jnp.float32))

    def body(h_hbm, src_hbm, dst_hbm, zrow_hbm, zdeg_hbm, *refs):
        if compute_deg:
            out_hbm, deg_hbm, srcv, dstv, *rest = refs
            *rest, ones, dacc = rest
        else:
            out_hbm, srcv, dstv, *rest = refs
        rows = tuple(rest[:NBUF])
        acc = rest[NBUF]
        semg = tuple(rest[NBUF + 1:NBUF + 1 + NBUF])
        cid = lax.axis_index("c")
        sid = lax.axis_index("s")
        row0 = pl.multiple_of(sid * RPT, 8)
        pltpu.sync_copy(zrow_hbm, acc.at[pl.ds(row0, RPT)])
        if compute_deg:
            pltpu.sync_copy(zdeg_hbm, dacc.at[pl.ds(row0, RPT)])
            for j in range(CH // L):
                ones[pl.ds(j * L, L)] = jnp.ones((L,), jnp.float32)
        plsc.subcore_barrier()

        rows0, rows1 = rows
        semg0, semg1 = semg

        def scat(buf, j):
            pltpu.sync_copy(buf, acc.at[dstv.at[j]], add=True)
            if compute_deg:
                pltpu.sync_copy(ones, dacc.at[dstv.at[j]], add=True)

        def pair(t, carry, last):
            j0 = 2 * t
            pltpu.async_copy(h_hbm.at[srcv.at[j0 + 1]], rows1, semg1)
            pltpu.make_async_copy(h_hbm.at[srcv.at[j0]], rows0, semg0).wait()
            scat(rows0, j0)
            if not last:
                pltpu.async_copy(h_hbm.at[srcv.at[j0 + 2]], rows0, semg0)
            pltpu.make_async_copy(h_hbm.at[srcv.at[j0 + 1]], rows1, semg1).wait()
            scat(rows1, j0 + 1)
            return carry

        def run_edges(chunk0, nstages):
            for s in range(nstages):
                start = pl.multiple_of(chunk0 + s * SUP, 8)
                pltpu.sync_copy(src_hbm.at[pl.ds(start, SUP)], srcv)
                pltpu.sync_copy(dst_hbm.at[pl.ds(start, SUP)], dstv)
                pltpu.async_copy(h_hbm.at[srcv.at[0]], rows0, semg0)
                lax.fori_loop(0, SUP // 2 - 1,
                              functools.partial(pair, last=False), 0)
                pair(SUP // 2 - 1, 0, last=True)

        @pl.when(cid == 0)
        def _():
            run_edges(sid * CPT0, CPT0 // SUP)

        @pl.when(cid == 1)
        def _():
            run_edges(NS * CPT0 + sid * CPT1, CPT1 // SUP)
        plsc.subcore_barrier()
        pltpu.sync_copy(acc.at[pl.ds(row0, RPT)],
                        out_hbm.at[cid, pl.ds(row0, RPT)])
        if compute_deg:
            pltpu.sync_copy(dacc.at[pl.ds(row0, RPT)],
                            deg_hbm.at[cid, pl.ds(row0, RPT)])

    if not compute_deg:
        out_type = part
    return functools.partial(
        pl.kernel, mesh=mesh, out_type=out_type, scratch_types=scratch,
    )(body)


_sc_agg_deg = _make_sc_agg(True)
_sc_agg = _make_sc_agg(False)


def _make_dense(relu: bool):
    def body(h_ref, p_ref, dp_ref, ws_ref, wn_ref, b_ref, o_ref):
        h = h_ref[...]
        p = p_ref[...]
        z = p[0] + p[1]
        dp = dp_ref[...]
        deg = jnp.maximum(dp[0] + dp[1], 1.0)
        mean = z / deg
        out = (jnp.dot(h, ws_ref[...], preferred_element_type=jnp.float32,
                       precision=lax.Precision.HIGHEST)
               + jnp.dot(mean, wn_ref[...], preferred_element_type=jnp.float32,
                         precision=lax.Precision.HIGHEST)
               + b_ref[...])
        if relu:
            out = jnp.maximum(out, 0.0)
        o_ref[...] = out

    return pl.pallas_call(
        body,
        grid=(NPAD // RB,),
        in_specs=[
            pl.BlockSpec((RB, D), lambda i: (i, 0)),
            pl.BlockSpec((NC, RB, D), lambda i: (0, i, 0)),
            pl.BlockSpec((NC, RB, 1), lambda i: (0, i, 0)),
            pl.BlockSpec((D, D), lambda i: (0, 0)),
            pl.BlockSpec((D, D), lambda i: (0, 0)),
            pl.BlockSpec((1, D), lambda i: (0, 0)),
        ],
        out_specs=pl.BlockSpec((RB, D), lambda i: (i, 0)),
        out_shape=jax.ShapeDtypeStruct((NPAD, D), jnp.float32),
    )


_dense_relu = _make_dense(True)
_dense_lin = _make_dense(False)


def kernel(x, edge_index, W1_self, W1_neigh, b1, W2_self, W2_neigh, b2):
    src = edge_index[0].astype(jnp.int32)
    dst = edge_index[1].astype(jnp.int32)
    pad_e = EPAD - E
    src_r = jnp.concatenate(
        [src, jnp.full((pad_e,), N, jnp.int32)]).reshape(TOT_CH, CH)
    dst_r = jnp.concatenate(
        [dst, jnp.full((pad_e,), NPAD - 1, jnp.int32)]).reshape(TOT_CH, CH)
    x_pad = jnp.pad(x, ((0, NPAD - N), (0, 0)))
    zrow = jnp.zeros((RPT, D), jnp.float32)
    zdeg = jnp.zeros((RPT,), jnp.float32)

    P1, degP = _sc_agg_deg(x_pad, src_r, dst_r, zrow, zdeg)
    dp3 = degP[..., None]
    h1 = _dense_relu(x_pad, P1, dp3, W1_self, W1_neigh, b1.reshape(1, D))
    P2 = _sc_agg(h1, src_r, dst_r, zrow, zdeg)
    out = _dense_lin(h1, P2, dp3, W2_self, W2_neigh, b2.reshape(1, D))
    return out[:N]


# speedup vs baseline: 1.2651x; 1.2651x over previous
import functools

import jax
import jax.numpy as jnp
from jax import lax
from jax.experimental import pallas as pl
from jax.experimental.pallas import tpu as pltpu
from jax.experimental.pallas import tpu_sc as plsc

N = 10000
D = 128
E = 320000
NC, NS, L = 2, 16, 16
NW = NC * NS
NPAD = 10240
CH = 128
EPW_CH = 80
EPW = CH * EPW_CH
SUP = 40
NSUP = EPW_CH // SUP
NBUF = 2
EPAD = EPW * NW
RPT = NPAD // NS
RB = 1024


def _make_sc_agg(compute_deg: bool):
    mesh = plsc.VectorSubcoreMesh(core_axis_name="c", subcore_axis_name="s")
    part = jax.ShapeDtypeStruct((NC, NPAD, D), jnp.float32)
    out_type = [part]
    scratch = (
        [pltpu.VMEM((SUP, CH), jnp.int32)] * 2
        + [pltpu.VMEM((CH, D), jnp.float32)] * NBUF
        + [pltpu.VMEM_SHARED((NPAD, D), jnp.float32)]
        + [pltpu.SemaphoreType.DMA] * NBUF
    )
    if compute_deg:
        out_type.append(jax.ShapeDtypeStruct((NC, NPAD), jnp.float32))
        scratch.append(pltpu.VMEM((CH,), jnp.float32))
        scratch.append(pltpu.VMEM_SHARED((NPAD,), jnp.float32))

    def body(h_hbm, src_hbm, dst_hbm, zrow_hbm, zdeg_hbm, *refs):
        if compute_deg:
            out_hbm, deg_hbm, srcv, dstv, *rest = refs
            *rest, ones, dacc = rest
        else:
            out_hbm, srcv, dstv, *rest = refs
        rows = tuple(rest[:NBUF])
        acc = rest[NBUF]
        semg = tuple(rest[NBUF + 1:NBUF + 1 + NBUF])
        cid = lax.axis_index("c")
        sid = lax.axis_index("s")
        wid = sid * NC + cid
        row0 = pl.multiple_of(sid * RPT, 8)
        pltpu.sync_copy(zrow_hbm, acc.at[pl.ds(row0, RPT)])
        if compute_deg:
            pltpu.sync_copy(zdeg_hbm, dacc.at[pl.ds(row0, RPT)])
            for j in range(CH // L):
                ones[pl.ds(j * L, L)] = jnp.ones((L,), jnp.float32)
        plsc.subcore_barrier()

        rows0, rows1 = rows
        semg0, semg1 = semg

        def scat(buf, j):
            pltpu.sync_copy(buf, acc.at[dstv.at[j]], add=True)
            if compute_deg:
                pltpu.sync_copy(ones, dacc.at[dstv.at[j]], add=True)

        def pair(t, carry, last):
            j0 = 2 * t
            pltpu.async_copy(h_hbm.at[srcv.at[j0 + 1]], rows1, semg1)
            pltpu.make_async_copy(h_hbm.at[srcv.at[j0]], rows0, semg0).wait()
            scat(rows0, j0)
            if not last:
                pltpu.async_copy(h_hbm.at[srcv.at[j0 + 2]], rows0, semg0)
            pltpu.make_async_copy(h_hbm.at[srcv.at[j0 + 1]], rows1, semg1).wait()
            scat(rows1, j0 + 1)
            return carry

        for s in range(NSUP):
            pltpu.sync_copy(src_hbm.at[wid, pl.ds(s * SUP, SUP)], srcv)
            pltpu.sync_copy(dst_hbm.at[wid, pl.ds(s * SUP, SUP)], dstv)
            pltpu.async_copy(h_hbm.at[srcv.at[0]], rows0, semg0)
            lax.fori_loop(0, SUP // 2 - 1,
                          functools.partial(pair, last=False), 0)
            pair(SUP // 2 - 1, 0, last=True)
        plsc.subcore_barrier()
        pltpu.sync_copy(acc.at[pl.ds(row0, RPT)],
                        out_hbm.at[cid, pl.ds(row0, RPT)])
        if compute_deg:
            pltpu.sync_copy(dacc.at[pl.ds(row0, RPT)],
                            deg_hbm.at[cid, pl.ds(row0, RPT)])

    if not compute_deg:
        out_type = part
    return functools.partial(
        pl.kernel, mesh=mesh, out_type=out_type, scratch_types=scratch,
    )(body)


_sc_agg_deg = _make_sc_agg(True)
_sc_agg = _make_sc_agg(False)


def _make_dense(relu: bool):
    def body(h_ref, p_ref, dp_ref, ws_ref, wn_ref, b_ref, o_ref):
        h = h_ref[...]
        p = p_ref[...]
        z = p[0] + p[1]
        dp = dp_ref[...]
        deg = jnp.maximum(dp[0] + dp[1], 1.0)
        mean = z / deg
        out = (jnp.dot(h, ws_ref[...], preferred_element_type=jnp.float32,
                       precision=lax.Precision.HIGHEST)
               + jnp.dot(mean, wn_ref[...], preferred_element_type=jnp.float32,
                         precision=lax.Precision.HIGHEST)
               + b_ref[...])
        if relu:
            out = jnp.maximum(out, 0.0)
        o_ref[...] = out

    return pl.pallas_call(
        body,
        grid=(NPAD // RB,),
        in_specs=[
            pl.BlockSpec((RB, D), lambda i: (i, 0)),
            pl.BlockSpec((NC, RB, D), lambda i: (0, i, 0)),
            pl.BlockSpec((NC, RB, 1), lambda i: (0, i, 0)),
            pl.BlockSpec((D, D), lambda i: (0, 0)),
            pl.BlockSpec((D, D), lambda i: (0, 0)),
            pl.BlockSpec((1, D), lambda i: (0, 0)),
        ],
        out_specs=pl.BlockSpec((RB, D), lambda i: (i, 0)),
        out_shape=jax.ShapeDtypeStruct((NPAD, D), jnp.float32),
    )


_dense_relu = _make_dense(True)
_dense_lin = _make_dense(False)


def kernel(x, edge_index, W1_self, W1_neigh, b1, W2_self, W2_neigh, b2):
    src = edge_index[0].astype(jnp.int32)
    dst = edge_index[1].astype(jnp.int32)
    pad_e = EPAD - E
    src_r = jnp.concatenate(
        [src, jnp.full((pad_e,), N, jnp.int32)]).reshape(NW, EPW_CH, CH)
    dst_r = jnp.concatenate(
        [dst, jnp.full((pad_e,), NPAD - 1, jnp.int32)]).reshape(NW, EPW_CH, CH)
    x_pad = jnp.pad(x, ((0, NPAD - N), (0, 0)))
    zrow = jnp.zeros((RPT, D), jnp.float32)
    zdeg = jnp.zeros((RPT,), jnp.float32)

    P1, degP = _sc_agg_deg(x_pad, src_r, dst_r, zrow, zdeg)
    dp3 = degP[..., None]
    h1 = _dense_relu(x_pad, P1, dp3, W1_self, W1_neigh, b1.reshape(1, D))
    P2 = _sc_agg(h1, src_r, dst_r, zrow, zdeg)
    out = _dense_lin(h1, P2, dp3, W2_self, W2_neigh, b2.reshape(1, D))
    return out[:N]
